# baseline (speedup 1.0000x reference)
import jax
import jax.numpy as jnp
from jax import lax
from jax.experimental import pallas as pl
from jax.experimental.pallas import tpu as pltpu
from jax.experimental.pallas import tpu_sc as plsc

N_NODES = 10000
N_EDGES = 320000
D_IN = 128
D_HID = 128
D_OUT = 64

N_TILES = 16
N_PAD = 10112
ROWS_PER_TILE = N_PAD // N_TILES
STEP = 128
TOT_STEPS = 2560
E_PAD = TOT_STEPS * STEP
DEG_SPLIT = 80
ROW_BLK = 1000


def _make_sc_aggregate(width, with_deg, edge_split, s_t, ph, nbuf, sdepth):
  n_ph = s_t // ph
  out_type = [jax.ShapeDtypeStruct((N_PAD, width), jnp.float32),
              jax.ShapeDtypeStruct((N_PAD, width), jnp.float32)]
  scratch = [pltpu.VMEM((ph, STEP), jnp.int32),
             pltpu.VMEM((ph, STEP), jnp.int32),
             pltpu.VMEM((nbuf, STEP, width), jnp.float32),
             pltpu.VMEM_SHARED((N_PAD, width), jnp.float32),
             pltpu.SemaphoreType.DMA,
             pltpu.SemaphoreType.DMA]
  if with_deg:
    out_type += [jax.ShapeDtypeStruct((N_PAD, 16), jnp.float32),
                 jax.ShapeDtypeStruct((N_PAD, 16), jnp.float32)]
    scratch += [pltpu.VMEM((STEP, 16), jnp.float32),
                pltpu.VMEM_SHARED((N_PAD, 16), jnp.float32),
                pltpu.SemaphoreType.DMA]

  mesh = plsc.VectorSubcoreMesh(core_axis_name="c", subcore_axis_name="s")

  def body(*refs):
    if with_deg:
      (ytab, src_hbm, dst_hbm, zf_hbm, zd_hbm, ones_hbm,
       out_a, out_b, deg_a, deg_b,
       src_v, dst_v, rows_v, agg_sh, gsem, ssem, ones_v, deg_sh, dsem) = refs
    else:
      (ytab, src_hbm, dst_hbm, zf_hbm,
       out_a, out_b,
       src_v, dst_v, rows_v, agg_sh, gsem, ssem) = refs
    c = lax.axis_index("c")
    s = lax.axis_index("s")
    tile_rows = pl.ds(s * ROWS_PER_TILE, ROWS_PER_TILE)
    if edge_split:
      tab = ytab
      idx_base = (c * N_TILES + s) * s_t
    else:
      tab = ytab.at[c]
      idx_base = s * s_t

    pltpu.sync_copy(zf_hbm.at[tile_rows], agg_sh.at[tile_rows])
    if with_deg:
      pltpu.sync_copy(zd_hbm.at[tile_rows], deg_sh.at[tile_rows])
      pltpu.sync_copy(ones_hbm, ones_v)
    plsc.subcore_barrier()

    def do_phase(p, carry):
      pltpu.sync_copy(src_hbm.at[pl.ds(idx_base + p * ph, ph)], src_v)
      pltpu.sync_copy(dst_hbm.at[pl.ds(idx_base + p * ph, ph)], dst_v)
      if with_deg and not edge_split:
        cnt = jnp.where(c == 0, p * ph < DEG_SPLIT, p * ph >= DEG_SPLIT)

      for b in range(sdepth):
        pltpu.async_copy(tab.at[src_v.at[b]], rows_v.at[b], gsem)

      def step(j, carry2):
        buf = rows_v.at[j % nbuf]
        pltpu.make_async_copy(tab.at[src_v.at[j]], buf, gsem).wait()
        pltpu.async_copy(buf, agg_sh.at[dst_v.at[j]], ssem, add=True)
        if with_deg:
          if edge_split:
            pltpu.async_copy(ones_v, deg_sh.at[dst_v.at[j]], dsem, add=True)
          else:
            @pl.when(cnt)
            def _():
              pltpu.async_copy(ones_v, deg_sh.at[dst_v.at[j]], dsem, add=True)

        @pl.when(j >= sdepth)
        def _():
          pltpu.make_async_copy(rows_v.at[(j - sdepth) % nbuf],
                                agg_sh.at[dst_v.at[j - sdepth]], ssem).wait()
          if with_deg:
            if edge_split:
              pltpu.make_async_copy(ones_v, deg_sh.at[dst_v.at[j - sdepth]],
                                    dsem).wait()
            else:
              @pl.when(cnt)
              def _():
                pltpu.make_async_copy(ones_v, deg_sh.at[dst_v.at[j - sdepth]],
                                      dsem).wait()

        @pl.when(j + sdepth < ph)
        def _():
          pltpu.async_copy(tab.at[src_v.at[j + sdepth]],
                           rows_v.at[(j + sdepth) % nbuf], gsem)
        return carry2

      lax.fori_loop(0, ph, step, 0)
      for j in range(ph - sdepth, ph):
        pltpu.make_async_copy(rows_v.at[j % nbuf],
                              agg_sh.at[dst_v.at[j]], ssem).wait()
        if with_deg:
          if edge_split:
            pltpu.make_async_copy(ones_v, deg_sh.at[dst_v.at[j]], dsem).wait()
          else:
            @pl.when(cnt)
            def _():
              pltpu.make_async_copy(ones_v, deg_sh.at[dst_v.at[j]],
                                    dsem).wait()
      return carry

    lax.fori_loop(0, n_ph, do_phase, 0)
    plsc.subcore_barrier()

    @pl.when(c == 0)
    def _():
      pltpu.sync_copy(agg_sh.at[tile_rows], out_a.at[tile_rows])
      if with_deg:
        pltpu.sync_copy(deg_sh.at[tile_rows], deg_a.at[tile_rows])

    @pl.when(c == 1)
    def _():
      pltpu.sync_copy(agg_sh.at[tile_rows], out_b.at[tile_rows])
      if with_deg:
        pltpu.sync_copy(deg_sh.at[tile_rows], deg_b.at[tile_rows])

  return pl.kernel(body, out_type=out_type, mesh=mesh, scratch_types=scratch,
                   compiler_params=pltpu.CompilerParams(use_tc_tiling_on_sc=False))


def _tc_layer1(x, wcat):
  def body(x_ref, w_ref, ytab_ref, z_ref):
    y = jnp.dot(x_ref[...], w_ref[...], preferred_element_type=jnp.float32)
    ytab_ref[0] = y[:, :64]
    ytab_ref[1] = y[:, 64:128]
    z_ref[...] = y[:, 128:]

  return pl.pallas_call(
      body,
      grid=(N_NODES // ROW_BLK,),
      in_specs=[pl.BlockSpec((ROW_BLK, D_IN), lambda i: (i, 0)),
                pl.BlockSpec((D_IN, 2 * D_HID), lambda i: (0, 0))],
      out_specs=[pl.BlockSpec((2, ROW_BLK, 64), lambda i: (0, i, 0)),
                 pl.BlockSpec((ROW_BLK, D_HID), lambda i: (i, 0))],
      out_shape=[jax.ShapeDtypeStruct((2, N_NODES, 64), jnp.float32),
                 jax.ShapeDtypeStruct((N_NODES, D_HID), jnp.float32)],
  )(x, wcat)


def _tc_layer2(agg_a, agg_b, deg_a, deg_b, z1, bl1, wcat, bl2):
  def body(a_ref, b_ref, da_ref, db_ref, z1_ref, bl1_ref, w_ref, bl2_ref,
           y_ref, z2_ref):
    deg = da_ref[:, 0:1] + db_ref[:, 0:1]
    inv = 1.0 / jnp.maximum(deg, 1.0)
    mean = jnp.concatenate([a_ref[...], b_ref[...]], axis=1) * inv
    h = jnp.maximum(mean + bl1_ref[...] + z1_ref[...], 0.0)
    y2 = jnp.dot(h, w_ref[...], preferred_element_type=jnp.float32)
    y_ref[...] = y2[:, :D_OUT]
    z2_ref[...] = y2[:, D_OUT:] + bl2_ref[...]

  return pl.pallas_call(
      body,
      grid=(N_NODES // ROW_BLK,),
      in_specs=[pl.BlockSpec((ROW_BLK, 64), lambda i: (i, 0)),
                pl.BlockSpec((ROW_BLK, 64), lambda i: (i, 0)),
                pl.BlockSpec((ROW_BLK, 16), lambda i: (i, 0)),
                pl.BlockSpec((ROW_BLK, 16), lambda i: (i, 0)),
                pl.BlockSpec((ROW_BLK, D_HID), lambda i: (i, 0)),
                pl.BlockSpec((1, D_HID), lambda i: (0, 0)),
                pl.BlockSpec((D_HID, 2 * D_OUT), lambda i: (0, 0)),
                pl.BlockSpec((1, D_OUT), lambda i: (0, 0))],
      out_specs=[pl.BlockSpec((ROW_BLK, D_OUT), lambda i: (i, 0)),
                 pl.BlockSpec((ROW_BLK, D_OUT), lambda i: (i, 0))],
      out_shape=[jax.ShapeDtypeStruct((N_NODES, D_OUT), jnp.float32),
                 jax.ShapeDtypeStruct((N_NODES, D_OUT), jnp.float32)],
  )(agg_a, agg_b, deg_a, deg_b, z1, bl1, wcat, bl2)


def _tc_out(agg_a, agg_b, deg_a, deg_b, z2):
  def body(a_ref, b_ref, da_ref, db_ref, z2_ref, o_ref):
    deg = da_ref[:, 0:1] + db_ref[:, 0:1]
    inv = 1.0 / jnp.maximum(deg, 1.0)
    o = (a_ref[...] + b_ref[...]) * inv + z2_ref[...]
    m = jnp.max(o, axis=1, keepdims=True)
    e = jnp.exp(o - m)
    o_ref[...] = o - m - jnp.log(jnp.sum(e, axis=1, keepdims=True))

  return pl.pallas_call(
      body,
      grid=(N_NODES // ROW_BLK,),
      in_specs=[pl.BlockSpec((ROW_BLK, D_OUT), lambda i: (i, 0)),
                pl.BlockSpec((ROW_BLK, D_OUT), lambda i: (i, 0)),
                pl.BlockSpec((ROW_BLK, 16), lambda i: (i, 0)),
                pl.BlockSpec((ROW_BLK, 16), lambda i: (i, 0)),
                pl.BlockSpec((ROW_BLK, D_OUT), lambda i: (i, 0))],
      out_specs=pl.BlockSpec((ROW_BLK, D_OUT), lambda i: (i, 0)),
      out_shape=jax.ShapeDtypeStruct((N_NODES, D_OUT), jnp.float32),
  )(agg_a, agg_b, deg_a, deg_b, z2)


_sc_agg1 = _make_sc_aggregate(64, with_deg=True, edge_split=False,
                              s_t=160, ph=40, nbuf=8, sdepth=4)
_sc_agg2 = _make_sc_aggregate(64, with_deg=False, edge_split=True,
                              s_t=80, ph=80, nbuf=8, sdepth=4)


def kernel(x, edge_index, Wl1, bl1, Wr1, Wl2, bl2, Wr2):
  src = edge_index[0].astype(jnp.int32)
  dst = edge_index[1].astype(jnp.int32)
  pad = E_PAD - N_EDGES
  src2d = jnp.concatenate([src, jnp.zeros((pad,), jnp.int32)]).reshape(-1, STEP)
  dst2d = jnp.concatenate(
      [dst, jnp.full((pad,), N_NODES, jnp.int32)]).reshape(-1, STEP)

  zf64 = jnp.zeros((N_PAD, 64), jnp.float32)
  zd = jnp.zeros((N_PAD, 16), jnp.float32)
  ones = jnp.ones((STEP, 16), jnp.float32)

  w1cat = jnp.concatenate([Wl1.T, Wr1.T], axis=1)
  w2cat = jnp.concatenate([Wl2.T, Wr2.T], axis=1)

  ytab1, z1 = _tc_layer1(x, w1cat)
  agg1a, agg1b, dega, degb = _sc_agg1(ytab1, src2d, dst2d, zf64, zd, ones)
  ytab2, z2 = _tc_layer2(agg1a, agg1b, dega, degb,
                         z1, bl1.reshape(1, -1), w2cat, bl2.reshape(1, -1))
  agg2a, agg2b = _sc_agg2(ytab2, src2d, dst2d, zf64)
  return _tc_out(agg2a, agg2b, dega, degb, z2)


# speedup vs baseline: 1.0007x; 1.0007x over previous
import jax
import jax.numpy as jnp
from jax import lax
from jax.experimental import pallas as pl
from jax.experimental.pallas import tpu as pltpu
from jax.experimental.pallas import tpu_sc as plsc

N_NODES = 10000
N_EDGES = 320000
D_IN = 128
D_HID = 128
D_OUT = 64

N_TILES = 16
N_PAD = 10112
ROWS_PER_TILE = N_PAD // N_TILES
STEP = 128
TOT_STEPS = 2560
E_PAD = TOT_STEPS * STEP
DEG_SPLIT = 80
ROW_BLK = 1000


def _make_sc_aggregate(width, with_deg, edge_split, s_t, ph, nbuf, sdepth):
  n_ph = s_t // ph
  out_type = [jax.ShapeDtypeStruct((N_PAD, width), jnp.float32),
              jax.ShapeDtypeStruct((N_PAD, width), jnp.float32)]
  scratch = [pltpu.VMEM((ph, STEP), jnp.int32),
             pltpu.VMEM((ph, STEP), jnp.int32),
             pltpu.VMEM((nbuf, STEP, width), jnp.float32),
             pltpu.VMEM_SHARED((N_PAD, width), jnp.float32),
             pltpu.SemaphoreType.DMA,
             pltpu.SemaphoreType.DMA]
  if with_deg:
    out_type += [jax.ShapeDtypeStruct((N_PAD, 16), jnp.float32),
                 jax.ShapeDtypeStruct((N_PAD, 16), jnp.float32)]
    scratch += [pltpu.VMEM((STEP, 16), jnp.float32),
                pltpu.VMEM_SHARED((N_PAD, 16), jnp.float32),
                pltpu.SemaphoreType.DMA]

  mesh = plsc.VectorSubcoreMesh(core_axis_name="c", subcore_axis_name="s")

  def body(*refs):
    if with_deg:
      (ytab, src_hbm, dst_hbm, zf_hbm, zd_hbm, ones_hbm,
       out_a, out_b, deg_a, deg_b,
       src_v, dst_v, rows_v, agg_sh, gsem, ssem, ones_v, deg_sh, dsem) = refs
    else:
      (ytab, src_hbm, dst_hbm, zf_hbm,
       out_a, out_b,
       src_v, dst_v, rows_v, agg_sh, gsem, ssem) = refs
    c = lax.axis_index("c")
    s = lax.axis_index("s")
    tile_rows = pl.ds(s * ROWS_PER_TILE, ROWS_PER_TILE)
    if edge_split:
      tab = ytab
      idx_base = (c * N_TILES + s) * s_t
    else:
      tab = ytab.at[c]
      idx_base = s * s_t

    pltpu.sync_copy(zf_hbm.at[tile_rows], agg_sh.at[tile_rows])
    if with_deg:
      pltpu.sync_copy(zd_hbm.at[tile_rows], deg_sh.at[tile_rows])
      pltpu.sync_copy(ones_hbm, ones_v)
    plsc.subcore_barrier()

    def do_phase(p, carry):
      pltpu.sync_copy(src_hbm.at[pl.ds(idx_base + p * ph, ph)], src_v)
      pltpu.sync_copy(dst_hbm.at[pl.ds(idx_base + p * ph, ph)], dst_v)
      if with_deg and not edge_split:
        cnt = jnp.where(c == 0, p * ph < DEG_SPLIT, p * ph >= DEG_SPLIT)

      for b in range(sdepth):
        pltpu.async_copy(tab.at[src_v.at[b]], rows_v.at[b], gsem)

      def step(j, carry2):
        buf = rows_v.at[j % nbuf]
        pltpu.make_async_copy(tab.at[src_v.at[j]], buf, gsem).wait()
        pltpu.async_copy(buf, agg_sh.at[dst_v.at[j]], ssem, add=True)
        if with_deg:
          if edge_split:
            pltpu.async_copy(ones_v, deg_sh.at[dst_v.at[j]], dsem, add=True)
          else:
            @pl.when(cnt)
            def _():
              pltpu.async_copy(ones_v, deg_sh.at[dst_v.at[j]], dsem, add=True)

        @pl.when(j >= sdepth)
        def _():
          pltpu.make_async_copy(rows_v.at[(j - sdepth) % nbuf],
                                agg_sh.at[dst_v.at[j - sdepth]], ssem).wait()
          if with_deg:
            if edge_split:
              pltpu.make_async_copy(ones_v, deg_sh.at[dst_v.at[j - sdepth]],
                                    dsem).wait()
            else:
              @pl.when(cnt)
              def _():
                pltpu.make_async_copy(ones_v, deg_sh.at[dst_v.at[j - sdepth]],
                                      dsem).wait()

        @pl.when(j + sdepth < ph)
        def _():
          pltpu.async_copy(tab.at[src_v.at[j + sdepth]],
                           rows_v.at[(j + sdepth) % nbuf], gsem)
        return carry2

      lax.fori_loop(0, ph, step, 0)
      for j in range(ph - sdepth, ph):
        pltpu.make_async_copy(rows_v.at[j % nbuf],
                              agg_sh.at[dst_v.at[j]], ssem).wait()
        if with_deg:
          if edge_split:
            pltpu.make_async_copy(ones_v, deg_sh.at[dst_v.at[j]], dsem).wait()
          else:
            @pl.when(cnt)
            def _():
              pltpu.make_async_copy(ones_v, deg_sh.at[dst_v.at[j]],
                                    dsem).wait()
      return carry

    lax.fori_loop(0, n_ph, do_phase, 0)
    plsc.subcore_barrier()

    @pl.when(c == 0)
    def _():
      pltpu.sync_copy(agg_sh.at[tile_rows], out_a.at[tile_rows])
      if with_deg:
        pltpu.sync_copy(deg_sh.at[tile_rows], deg_a.at[tile_rows])

    @pl.when(c == 1)
    def _():
      pltpu.sync_copy(agg_sh.at[tile_rows], out_b.at[tile_rows])
      if with_deg:
        pltpu.sync_copy(deg_sh.at[tile_rows], deg_b.at[tile_rows])

  return pl.kernel(body, out_type=out_type, mesh=mesh, scratch_types=scratch,
                   compiler_params=pltpu.CompilerParams(use_tc_tiling_on_sc=False))


def _tc_layer1(x, wcat):
  def body(x_ref, w_ref, ytab_ref, z_ref):
    y = jnp.dot(x_ref[...], w_ref[...], preferred_element_type=jnp.float32)
    ytab_ref[0] = y[:, :64]
    ytab_ref[1] = y[:, 64:128]
    z_ref[...] = y[:, 128:]

  return pl.pallas_call(
      body,
      grid=(N_NODES // ROW_BLK,),
      in_specs=[pl.BlockSpec((ROW_BLK, D_IN), lambda i: (i, 0)),
                pl.BlockSpec((D_IN, 2 * D_HID), lambda i: (0, 0))],
      out_specs=[pl.BlockSpec((2, ROW_BLK, 64), lambda i: (0, i, 0)),
                 pl.BlockSpec((ROW_BLK, D_HID), lambda i: (i, 0))],
      out_shape=[jax.ShapeDtypeStruct((2, N_NODES, 64), jnp.float32),
                 jax.ShapeDtypeStruct((N_NODES, D_HID), jnp.float32)],
  )(x, wcat)


def _tc_layer2(agg_a, agg_b, deg_a, deg_b, z1, bl1, wcat, bl2):
  def body(a_ref, b_ref, da_ref, db_ref, z1_ref, bl1_ref, w_ref, bl2_ref,
           y_ref, z2_ref):
    deg = da_ref[:, 0:1] + db_ref[:, 0:1]
    inv = 1.0 / jnp.maximum(deg, 1.0)
    mean = jnp.concatenate([a_ref[...], b_ref[...]], axis=1) * inv
    h = jnp.maximum(mean + bl1_ref[...] + z1_ref[...], 0.0)
    y2 = jnp.dot(h, w_ref[...], preferred_element_type=jnp.float32)
    y_ref[...] = y2[:, :D_OUT]
    z2_ref[...] = y2[:, D_OUT:] + bl2_ref[...]

  return pl.pallas_call(
      body,
      grid=(N_NODES // ROW_BLK,),
      in_specs=[pl.BlockSpec((ROW_BLK, 64), lambda i: (i, 0)),
                pl.BlockSpec((ROW_BLK, 64), lambda i: (i, 0)),
                pl.BlockSpec((ROW_BLK, 16), lambda i: (i, 0)),
                pl.BlockSpec((ROW_BLK, 16), lambda i: (i, 0)),
                pl.BlockSpec((ROW_BLK, D_HID), lambda i: (i, 0)),
                pl.BlockSpec((1, D_HID), lambda i: (0, 0)),
                pl.BlockSpec((D_HID, 2 * D_OUT), lambda i: (0, 0)),
                pl.BlockSpec((1, D_OUT), lambda i: (0, 0))],
      out_specs=[pl.BlockSpec((ROW_BLK, D_OUT), lambda i: (i, 0)),
                 pl.BlockSpec((ROW_BLK, D_OUT), lambda i: (i, 0))],
      out_shape=[jax.ShapeDtypeStruct((N_NODES, D_OUT), jnp.float32),
                 jax.ShapeDtypeStruct((N_NODES, D_OUT), jnp.float32)],
  )(agg_a, agg_b, deg_a, deg_b, z1, bl1, wcat, bl2)


def _tc_out(agg_a, agg_b, deg_a, deg_b, z2):
  def body(a_ref, b_ref, da_ref, db_ref, z2_ref, o_ref):
    deg = da_ref[:, 0:1] + db_ref[:, 0:1]
    inv = 1.0 / jnp.maximum(deg, 1.0)
    o = (a_ref[...] + b_ref[...]) * inv + z2_ref[...]
    m = jnp.max(o, axis=1, keepdims=True)
    e = jnp.exp(o - m)
    o_ref[...] = o - m - jnp.log(jnp.sum(e, axis=1, keepdims=True))

  return pl.pallas_call(
      body,
      grid=(N_NODES // ROW_BLK,),
      in_specs=[pl.BlockSpec((ROW_BLK, D_OUT), lambda i: (i, 0)),
                pl.BlockSpec((ROW_BLK, D_OUT), lambda i: (i, 0)),
                pl.BlockSpec((ROW_BLK, 16), lambda i: (i, 0)),
                pl.BlockSpec((ROW_BLK, 16), lambda i: (i, 0)),
                pl.BlockSpec((ROW_BLK, D_OUT), lambda i: (i, 0))],
      out_specs=pl.BlockSpec((ROW_BLK, D_OUT), lambda i: (i, 0)),
      out_shape=jax.ShapeDtypeStruct((N_NODES, D_OUT), jnp.float32),
  )(agg_a, agg_b, deg_a, deg_b, z2)


_sc_agg1 = _make_sc_aggregate(64, with_deg=True, edge_split=False,
                              s_t=160, ph=40, nbuf=8, sdepth=4)
_sc_agg2 = _make_sc_aggregate(64, with_deg=False, edge_split=True,
                              s_t=80, ph=80, nbuf=8, sdepth=4)


def kernel(x, edge_index, Wl1, bl1, Wr1, Wl2, bl2, Wr2):
  src = edge_index[0].astype(jnp.int32)
  dst = edge_index[1].astype(jnp.int32)
  pad = E_PAD - N_EDGES
  pad_dst = N_NODES + (jnp.arange(pad, dtype=jnp.int32) % (N_PAD - N_NODES))
  src2d = jnp.concatenate([src, jnp.zeros((pad,), jnp.int32)]).reshape(-1, STEP)
  dst2d = jnp.concatenate([dst, pad_dst]).reshape(-1, STEP)

  zf64 = jnp.zeros((N_PAD, 64), jnp.float32)
  zd = jnp.zeros((N_PAD, 16), jnp.float32)
  ones = jnp.ones((STEP, 16), jnp.float32)

  w1cat = jnp.concatenate([Wl1.T, Wr1.T], axis=1)
  w2cat = jnp.concatenate([Wl2.T, Wr2.T], axis=1)

  ytab1, z1 = _tc_layer1(x, w1cat)
  agg1a, agg1b, dega, degb = _sc_agg1(ytab1, src2d, dst2d, zf64, zd, ones)
  ytab2, z2 = _tc_layer2(agg1a, agg1b, dega, degb,
                         z1, bl1.reshape(1, -1), w2cat, bl2.reshape(1, -1))
  agg2a, agg2b = _sc_agg2(ytab2, src2d, dst2d, zf64)
  return _tc_out(agg2a, agg2b, dega, degb, z2)


# speedup vs baseline: 2.4623x; 2.4607x over previous
import jax
import jax.numpy as jnp
from jax import lax
from jax.experimental import pallas as pl
from jax.experimental.pallas import tpu as pltpu
from jax.experimental.pallas import tpu_sc as plsc

N_NODES = 10000
N_EDGES = 320000
D_IN = 128
D_HID = 128
D_OUT = 64

N_TILES = 16
N_PAD = 10112
ROWS_PER_TILE = N_PAD // N_TILES
STEP = 128
TOT_STEPS = 2560
E_PAD = TOT_STEPS * STEP
DEG_SPLIT = 80
ROW_BLK = 1000


def _make_sc_aggregate(width, with_deg, edge_split, s_t, ph, nbuf, sdepth):
  n_ph = s_t // ph
  out_type = [jax.ShapeDtypeStruct((N_PAD, width), jnp.float32),
              jax.ShapeDtypeStruct((N_PAD, width), jnp.float32)]
  scratch = [pltpu.VMEM((ph, STEP), jnp.int32),
             pltpu.VMEM((ph, STEP), jnp.int32),
             pltpu.VMEM((nbuf, STEP, width), jnp.float32),
             pltpu.VMEM_SHARED((N_PAD, width), jnp.float32),
             pltpu.SemaphoreType.DMA,
             pltpu.SemaphoreType.DMA]
  if with_deg:
    out_type += [jax.ShapeDtypeStruct((N_PAD, 16), jnp.float32),
                 jax.ShapeDtypeStruct((N_PAD, 16), jnp.float32)]
    scratch += [pltpu.VMEM((STEP, 16), jnp.float32),
                pltpu.VMEM_SHARED((N_PAD, 16), jnp.float32),
                pltpu.SemaphoreType.DMA]

  mesh = plsc.VectorSubcoreMesh(core_axis_name="c", subcore_axis_name="s")

  def body(*refs):
    if with_deg:
      (ytab, src_hbm, dst_hbm, zf_hbm, zd_hbm, ones_hbm,
       out_a, out_b, deg_a, deg_b,
       src_v, dst_v, rows_v, agg_sh, gsem, ssem, ones_v, deg_sh, dsem) = refs
    else:
      (ytab, src_hbm, dst_hbm, zf_hbm,
       out_a, out_b,
       src_v, dst_v, rows_v, agg_sh, gsem, ssem) = refs
    c = lax.axis_index("c")
    s = lax.axis_index("s")
    tile_rows = pl.ds(s * ROWS_PER_TILE, ROWS_PER_TILE)
    if edge_split:
      tab = ytab
      idx_base = (c * N_TILES + s) * s_t
    else:
      tab = ytab.at[c]
      idx_base = s * s_t

    pltpu.sync_copy(zf_hbm.at[tile_rows], agg_sh.at[tile_rows])
    if with_deg:
      pltpu.sync_copy(zd_hbm.at[tile_rows], deg_sh.at[tile_rows])
      pltpu.sync_copy(ones_hbm, ones_v)
    plsc.subcore_barrier()

    def do_phase(p, carry):
      pltpu.sync_copy(src_hbm.at[pl.ds(idx_base + p * ph, ph)], src_v)
      pltpu.sync_copy(dst_hbm.at[pl.ds(idx_base + p * ph, ph)], dst_v)
      if with_deg and not edge_split:
        cnt = jnp.where(c == 0, p * ph < DEG_SPLIT, p * ph >= DEG_SPLIT)

      for b in range(sdepth):
        pltpu.async_copy(tab.at[src_v.at[b]], rows_v.at[b], gsem)

      def step(j, carry2):
        buf = rows_v.at[j % nbuf]
        pltpu.make_async_copy(tab.at[src_v.at[j]], buf, gsem).wait()
        pltpu.async_copy(buf, agg_sh.at[dst_v.at[j]], ssem, add=True)
        if with_deg:
          if edge_split:
            pltpu.async_copy(ones_v, deg_sh.at[dst_v.at[j]], dsem, add=True)
          else:
            @pl.when(cnt)
            def _():
              pltpu.async_copy(ones_v, deg_sh.at[dst_v.at[j]], dsem, add=True)

        @pl.when(j >= sdepth)
        def _():
          pltpu.make_async_copy(rows_v.at[(j - sdepth) % nbuf],
                                agg_sh.at[dst_v.at[j - sdepth]], ssem).wait()
          if with_deg:
            if edge_split:
              pltpu.make_async_copy(ones_v, deg_sh.at[dst_v.at[j - sdepth]],
                                    dsem).wait()
            else:
              @pl.when(cnt)
              def _():
                pltpu.make_async_copy(ones_v, deg_sh.at[dst_v.at[j - sdepth]],
                                      dsem).wait()

        @pl.when(j + sdepth < ph)
        def _():
          pltpu.async_copy(tab.at[src_v.at[j + sdepth]],
                           rows_v.at[(j + sdepth) % nbuf], gsem)
        return carry2

      lax.fori_loop(0, ph, step, 0)
      for j in range(ph - sdepth, ph):
        pltpu.make_async_copy(rows_v.at[j % nbuf],
                              agg_sh.at[dst_v.at[j]], ssem).wait()
        if with_deg:
          if edge_split:
            pltpu.make_async_copy(ones_v, deg_sh.at[dst_v.at[j]], dsem).wait()
          else:
            @pl.when(cnt)
            def _():
              pltpu.make_async_copy(ones_v, deg_sh.at[dst_v.at[j]],
                                    dsem).wait()
      return carry

    lax.fori_loop(0, n_ph, do_phase, 0)
    plsc.subcore_barrier()

    @pl.when(c == 0)
    def _():
      pltpu.sync_copy(agg_sh.at[tile_rows], out_a.at[tile_rows])
      if with_deg:
        pltpu.sync_copy(deg_sh.at[tile_rows], deg_a.at[tile_rows])

    @pl.when(c == 1)
    def _():
      pltpu.sync_copy(agg_sh.at[tile_rows], out_b.at[tile_rows])
      if with_deg:
        pltpu.sync_copy(deg_sh.at[tile_rows], deg_b.at[tile_rows])

  return pl.kernel(body, out_type=out_type, mesh=mesh, scratch_types=scratch,
                   compiler_params=pltpu.CompilerParams(use_tc_tiling_on_sc=False))


def _tc_layer1(x, wcat):
  def body(x_ref, w_ref, ytab_ref, z_ref):
    y = jnp.dot(x_ref[...], w_ref[...], preferred_element_type=jnp.float32)
    ytab_ref[0] = y[:, :64]
    ytab_ref[1] = y[:, 64:128]
    z_ref[...] = y[:, 128:]

  return pl.pallas_call(
      body,
      grid=(N_NODES // ROW_BLK,),
      in_specs=[pl.BlockSpec((ROW_BLK, D_IN), lambda i: (i, 0)),
                pl.BlockSpec((D_IN, 2 * D_HID), lambda i: (0, 0))],
      out_specs=[pl.BlockSpec((2, ROW_BLK, 64), lambda i: (0, i, 0)),
                 pl.BlockSpec((ROW_BLK, D_HID), lambda i: (i, 0))],
      out_shape=[jax.ShapeDtypeStruct((2, N_NODES, 64), jnp.float32),
                 jax.ShapeDtypeStruct((N_NODES, D_HID), jnp.float32)],
  )(x, wcat)


def _tc_layer2(agg_a, agg_b, deg_a, deg_b, z1, bl1, wcat, bl2):
  def body(a_ref, b_ref, da_ref, db_ref, z1_ref, bl1_ref, w_ref, bl2_ref,
           y_ref, z2_ref):
    deg = da_ref[:, 0:1] + db_ref[:, 0:1]
    inv = 1.0 / jnp.maximum(deg, 1.0)
    mean = jnp.concatenate([a_ref[...], b_ref[...]], axis=1) * inv
    h = jnp.maximum(mean + bl1_ref[...] + z1_ref[...], 0.0)
    y2 = jnp.dot(h, w_ref[...], preferred_element_type=jnp.float32)
    y_ref[...] = y2[:, :D_OUT]
    z2_ref[...] = y2[:, D_OUT:] + bl2_ref[...]

  return pl.pallas_call(
      body,
      grid=(N_NODES // ROW_BLK,),
      in_specs=[pl.BlockSpec((ROW_BLK, 64), lambda i: (i, 0)),
                pl.BlockSpec((ROW_BLK, 64), lambda i: (i, 0)),
                pl.BlockSpec((ROW_BLK, 16), lambda i: (i, 0)),
                pl.BlockSpec((ROW_BLK, 16), lambda i: (i, 0)),
                pl.BlockSpec((ROW_BLK, D_HID), lambda i: (i, 0)),
                pl.BlockSpec((1, D_HID), lambda i: (0, 0)),
                pl.BlockSpec((D_HID, 2 * D_OUT), lambda i: (0, 0)),
                pl.BlockSpec((1, D_OUT), lambda i: (0, 0))],
      out_specs=[pl.BlockSpec((ROW_BLK, D_OUT), lambda i: (i, 0)),
                 pl.BlockSpec((ROW_BLK, D_OUT), lambda i: (i, 0))],
      out_shape=[jax.ShapeDtypeStruct((N_NODES, D_OUT), jnp.float32),
                 jax.ShapeDtypeStruct((N_NODES, D_OUT), jnp.float32)],
  )(agg_a, agg_b, deg_a, deg_b, z1, bl1, wcat, bl2)


def _tc_out(agg_a, agg_b, deg_a, deg_b, z2):
  def body(a_ref, b_ref, da_ref, db_ref, z2_ref, o_ref):
    deg = da_ref[:, 0:1] + db_ref[:, 0:1]
    inv = 1.0 / jnp.maximum(deg, 1.0)
    o = (a_ref[...] + b_ref[...]) * inv + z2_ref[...]
    m = jnp.max(o, axis=1, keepdims=True)
    e = jnp.exp(o - m)
    o_ref[...] = o - m - jnp.log(jnp.sum(e, axis=1, keepdims=True))

  return pl.pallas_call(
      body,
      grid=(N_NODES // ROW_BLK,),
      in_specs=[pl.BlockSpec((ROW_BLK, D_OUT), lambda i: (i, 0)),
                pl.BlockSpec((ROW_BLK, D_OUT), lambda i: (i, 0)),
                pl.BlockSpec((ROW_BLK, 16), lambda i: (i, 0)),
                pl.BlockSpec((ROW_BLK, 16), lambda i: (i, 0)),
                pl.BlockSpec((ROW_BLK, D_OUT), lambda i: (i, 0))],
      out_specs=pl.BlockSpec((ROW_BLK, D_OUT), lambda i: (i, 0)),
      out_shape=jax.ShapeDtypeStruct((N_NODES, D_OUT), jnp.float32),
  )(agg_a, agg_b, deg_a, deg_b, z2)


_sc_agg1 = _make_sc_aggregate(64, with_deg=True, edge_split=False,
                              s_t=160, ph=40, nbuf=8, sdepth=4)
_sc_agg2 = _make_sc_aggregate(64, with_deg=False, edge_split=True,
                              s_t=80, ph=80, nbuf=8, sdepth=4)


def kernel(x, edge_index, Wl1, bl1, Wr1, Wl2, bl2, Wr2):
  src = edge_index[0].astype(jnp.int32)
  dst = edge_index[1].astype(jnp.int32)
  pad = E_PAD - N_EDGES
  pad_ar = jnp.arange(pad, dtype=jnp.int32)
  pad_dst = N_NODES + pad_ar % (N_PAD - N_NODES)
  src2d = jnp.concatenate([src, pad_ar % N_NODES]).reshape(-1, STEP)
  dst2d = jnp.concatenate([dst, pad_dst]).reshape(-1, STEP)

  zf64 = jnp.zeros((N_PAD, 64), jnp.float32)
  zd = jnp.zeros((N_PAD, 16), jnp.float32)
  ones = jnp.ones((STEP, 16), jnp.float32)

  w1cat = jnp.concatenate([Wl1.T, Wr1.T], axis=1)
  w2cat = jnp.concatenate([Wl2.T, Wr2.T], axis=1)

  ytab1, z1 = _tc_layer1(x, w1cat)
  agg1a, agg1b, dega, degb = _sc_agg1(ytab1, src2d, dst2d, zf64, zd, ones)
  ytab2, z2 = _tc_layer2(agg1a, agg1b, dega, degb,
                         z1, bl1.reshape(1, -1), w2cat, bl2.reshape(1, -1))
  agg2a, agg2b = _sc_agg2(ytab2, src2d, dst2d, zf64)
  return _tc_out(agg2a, agg2b, dega, degb, z2)


# speedup vs baseline: 2.5013x; 1.0158x over previous
import jax
import jax.numpy as jnp
from jax import lax
from jax.experimental import pallas as pl
from jax.experimental.pallas import tpu as pltpu
from jax.experimental.pallas import tpu_sc as plsc

N_NODES = 10000
N_EDGES = 320000
D_IN = 128
D_HID = 128
D_OUT = 64

N_TILES = 16
N_PAD = 10112
ROWS_PER_TILE = N_PAD // N_TILES
STEP = 128
TOT_STEPS = 2560
E_PAD = TOT_STEPS * STEP
DEG_SPLIT = 80
ROW_BLK = 1000


def _make_sc_aggregate(width, with_deg, edge_split, s_t, ph, nbuf, sdepth):
  n_ph = s_t // ph
  out_type = [jax.ShapeDtypeStruct((N_PAD, width), jnp.float32),
              jax.ShapeDtypeStruct((N_PAD, width), jnp.float32)]
  scratch = [pltpu.VMEM((ph, STEP), jnp.int32),
             pltpu.VMEM((ph, STEP), jnp.int32),
             pltpu.VMEM((nbuf, STEP, width), jnp.float32),
             pltpu.VMEM_SHARED((N_PAD, width), jnp.float32),
             pltpu.SemaphoreType.DMA,
             pltpu.SemaphoreType.DMA]
  if with_deg:
    out_type += [jax.ShapeDtypeStruct((N_PAD, 16), jnp.float32),
                 jax.ShapeDtypeStruct((N_PAD, 16), jnp.float32)]
    scratch += [pltpu.VMEM((STEP, 16), jnp.float32),
                pltpu.VMEM_SHARED((N_PAD, 16), jnp.float32),
                pltpu.SemaphoreType.DMA]

  mesh = plsc.VectorSubcoreMesh(core_axis_name="c", subcore_axis_name="s")

  def body(*refs):
    if with_deg:
      (ytab, src_hbm, dst_hbm, zf_hbm, zd_hbm, ones_hbm,
       out_a, out_b, deg_a, deg_b,
       src_v, dst_v, rows_v, agg_sh, gsem, ssem, ones_v, deg_sh, dsem) = refs
    else:
      (ytab, src_hbm, dst_hbm, zf_hbm,
       out_a, out_b,
       src_v, dst_v, rows_v, agg_sh, gsem, ssem) = refs
    c = lax.axis_index("c")
    s = lax.axis_index("s")
    tile_rows = pl.ds(s * ROWS_PER_TILE, ROWS_PER_TILE)
    if edge_split:
      tab = ytab
      idx_base = (c * N_TILES + s) * s_t
    else:
      tab = ytab.at[c]
      idx_base = s * s_t

    pltpu.sync_copy(zf_hbm.at[tile_rows], agg_sh.at[tile_rows])
    if with_deg:
      pltpu.sync_copy(zd_hbm.at[tile_rows], deg_sh.at[tile_rows])
      pltpu.sync_copy(ones_hbm, ones_v)
    plsc.subcore_barrier()

    def do_phase(p, carry):
      pltpu.sync_copy(src_hbm.at[pl.ds(idx_base + p * ph, ph)], src_v)
      pltpu.sync_copy(dst_hbm.at[pl.ds(idx_base + p * ph, ph)], dst_v)
      if with_deg and not edge_split:
        cnt = jnp.where(c == 0, p * ph < DEG_SPLIT, p * ph >= DEG_SPLIT)

      for b in range(sdepth):
        pltpu.async_copy(tab.at[src_v.at[b]], rows_v.at[b], gsem)

      def step(j, carry2):
        buf = rows_v.at[j % nbuf]
        pltpu.make_async_copy(tab.at[src_v.at[j]], buf, gsem).wait()
        pltpu.async_copy(buf, agg_sh.at[dst_v.at[j]], ssem, add=True)
        if with_deg:
          if edge_split:
            pltpu.async_copy(ones_v, deg_sh.at[dst_v.at[j]], dsem, add=True)
          else:
            @pl.when(cnt)
            def _():
              pltpu.async_copy(ones_v, deg_sh.at[dst_v.at[j]], dsem, add=True)

        @pl.when(j >= sdepth)
        def _():
          pltpu.make_async_copy(rows_v.at[(j - sdepth) % nbuf],
                                agg_sh.at[dst_v.at[j - sdepth]], ssem).wait()
          if with_deg:
            if edge_split:
              pltpu.make_async_copy(ones_v, deg_sh.at[dst_v.at[j - sdepth]],
                                    dsem).wait()
            else:
              @pl.when(cnt)
              def _():
                pltpu.make_async_copy(ones_v, deg_sh.at[dst_v.at[j - sdepth]],
                                      dsem).wait()

        @pl.when(j + sdepth < ph)
        def _():
          pltpu.async_copy(tab.at[src_v.at[j + sdepth]],
                           rows_v.at[(j + sdepth) % nbuf], gsem)
        return carry2

      lax.fori_loop(0, ph, step, 0)
      for j in range(ph - sdepth, ph):
        pltpu.make_async_copy(rows_v.at[j % nbuf],
                              agg_sh.at[dst_v.at[j]], ssem).wait()
        if with_deg:
          if edge_split:
            pltpu.make_async_copy(ones_v, deg_sh.at[dst_v.at[j]], dsem).wait()
          else:
            @pl.when(cnt)
            def _():
              pltpu.make_async_copy(ones_v, deg_sh.at[dst_v.at[j]],
                                    dsem).wait()
      return carry

    lax.fori_loop(0, n_ph, do_phase, 0)
    plsc.subcore_barrier()

    @pl.when(c == 0)
    def _():
      pltpu.sync_copy(agg_sh.at[tile_rows], out_a.at[tile_rows])
      if with_deg:
        pltpu.sync_copy(deg_sh.at[tile_rows], deg_a.at[tile_rows])

    @pl.when(c == 1)
    def _():
      pltpu.sync_copy(agg_sh.at[tile_rows], out_b.at[tile_rows])
      if with_deg:
        pltpu.sync_copy(deg_sh.at[tile_rows], deg_b.at[tile_rows])

  return pl.kernel(body, out_type=out_type, mesh=mesh, scratch_types=scratch,
                   compiler_params=pltpu.CompilerParams(use_tc_tiling_on_sc=False))


def _tc_layer1(x, wcat):
  def body(x_ref, w_ref, ytab_ref, z_ref):
    y = jnp.dot(x_ref[...], w_ref[...], preferred_element_type=jnp.float32)
    ytab_ref[0] = y[:, :64]
    ytab_ref[1] = y[:, 64:128]
    z_ref[...] = y[:, 128:]

  return pl.pallas_call(
      body,
      grid=(N_NODES // ROW_BLK,),
      in_specs=[pl.BlockSpec((ROW_BLK, D_IN), lambda i: (i, 0)),
                pl.BlockSpec((D_IN, 2 * D_HID), lambda i: (0, 0))],
      out_specs=[pl.BlockSpec((2, ROW_BLK, 64), lambda i: (0, i, 0)),
                 pl.BlockSpec((ROW_BLK, D_HID), lambda i: (i, 0))],
      out_shape=[jax.ShapeDtypeStruct((2, N_NODES, 64), jnp.float32),
                 jax.ShapeDtypeStruct((N_NODES, D_HID), jnp.float32)],
  )(x, wcat)


def _tc_layer2(agg_a, agg_b, deg_a, deg_b, z1, bl1, wcat, bl2):
  def body(a_ref, b_ref, da_ref, db_ref, z1_ref, bl1_ref, w_ref, bl2_ref,
           y_ref, z2_ref):
    deg = da_ref[:, 0:1] + db_ref[:, 0:1]
    inv = 1.0 / jnp.maximum(deg, 1.0)
    mean = jnp.concatenate([a_ref[...], b_ref[...]], axis=1) * inv
    h = jnp.maximum(mean + bl1_ref[...] + z1_ref[...], 0.0)
    y2 = jnp.dot(h, w_ref[...], preferred_element_type=jnp.float32)
    y_ref[...] = y2[:, :D_OUT]
    z2_ref[...] = y2[:, D_OUT:] + bl2_ref[...]

  return pl.pallas_call(
      body,
      grid=(N_NODES // ROW_BLK,),
      in_specs=[pl.BlockSpec((ROW_BLK, 64), lambda i: (i, 0)),
                pl.BlockSpec((ROW_BLK, 64), lambda i: (i, 0)),
                pl.BlockSpec((ROW_BLK, 16), lambda i: (i, 0)),
                pl.BlockSpec((ROW_BLK, 16), lambda i: (i, 0)),
                pl.BlockSpec((ROW_BLK, D_HID), lambda i: (i, 0)),
                pl.BlockSpec((1, D_HID), lambda i: (0, 0)),
                pl.BlockSpec((D_HID, 2 * D_OUT), lambda i: (0, 0)),
                pl.BlockSpec((1, D_OUT), lambda i: (0, 0))],
      out_specs=[pl.BlockSpec((ROW_BLK, D_OUT), lambda i: (i, 0)),
                 pl.BlockSpec((ROW_BLK, D_OUT), lambda i: (i, 0))],
      out_shape=[jax.ShapeDtypeStruct((N_NODES, D_OUT), jnp.float32),
                 jax.ShapeDtypeStruct((N_NODES, D_OUT), jnp.float32)],
  )(agg_a, agg_b, deg_a, deg_b, z1, bl1, wcat, bl2)


def _tc_out(agg_a, agg_b, deg_a, deg_b, z2):
  def body(a_ref, b_ref, da_ref, db_ref, z2_ref, o_ref):
    deg = da_ref[:, 0:1] + db_ref[:, 0:1]
    inv = 1.0 / jnp.maximum(deg, 1.0)
    o = (a_ref[...] + b_ref[...]) * inv + z2_ref[...]
    m = jnp.max(o, axis=1, keepdims=True)
    e = jnp.exp(o - m)
    o_ref[...] = o - m - jnp.log(jnp.sum(e, axis=1, keepdims=True))

  return pl.pallas_call(
      body,
      grid=(N_NODES // ROW_BLK,),
      in_specs=[pl.BlockSpec((ROW_BLK, D_OUT), lambda i: (i, 0)),
                pl.BlockSpec((ROW_BLK, D_OUT), lambda i: (i, 0)),
                pl.BlockSpec((ROW_BLK, 16), lambda i: (i, 0)),
                pl.BlockSpec((ROW_BLK, 16), lambda i: (i, 0)),
                pl.BlockSpec((ROW_BLK, D_OUT), lambda i: (i, 0))],
      out_specs=pl.BlockSpec((ROW_BLK, D_OUT), lambda i: (i, 0)),
      out_shape=jax.ShapeDtypeStruct((N_NODES, D_OUT), jnp.float32),
  )(agg_a, agg_b, deg_a, deg_b, z2)


_sc_agg1 = _make_sc_aggregate(64, with_deg=True, edge_split=False,
                              s_t=160, ph=80, nbuf=6, sdepth=3)
_sc_agg2 = _make_sc_aggregate(64, with_deg=False, edge_split=True,
                              s_t=80, ph=80, nbuf=8, sdepth=4)


def kernel(x, edge_index, Wl1, bl1, Wr1, Wl2, bl2, Wr2):
  src = edge_index[0].astype(jnp.int32)
  dst = edge_index[1].astype(jnp.int32)
  pad = E_PAD - N_EDGES
  pad_ar = jnp.arange(pad, dtype=jnp.int32)
  pad_dst = N_NODES + pad_ar % (N_PAD - N_NODES)
  src2d = jnp.concatenate([src, pad_ar % N_NODES]).reshape(-1, STEP)
  dst2d = jnp.concatenate([dst, pad_dst]).reshape(-1, STEP)

  zf64 = jnp.zeros((N_PAD, 64), jnp.float32)
  zd = jnp.zeros((N_PAD, 16), jnp.float32)
  ones = jnp.ones((STEP, 16), jnp.float32)

  w1cat = jnp.concatenate([Wl1.T, Wr1.T], axis=1)
  w2cat = jnp.concatenate([Wl2.T, Wr2.T], axis=1)

  ytab1, z1 = _tc_layer1(x, w1cat)
  agg1a, agg1b, dega, degb = _sc_agg1(ytab1, src2d, dst2d, zf64, zd, ones)
  ytab2, z2 = _tc_layer2(agg1a, agg1b, dega, degb,
                         z1, bl1.reshape(1, -1), w2cat, bl2.reshape(1, -1))
  agg2a, agg2b = _sc_agg2(ytab2, src2d, dst2d, zf64)
  return _tc_out(agg2a, agg2b, dega, degb, z2)


# speedup vs baseline: 2.5629x; 1.0246x over previous
import jax
import jax.numpy as jnp
from jax import lax
from jax.experimental import pallas as pl
from jax.experimental.pallas import tpu as pltpu
from jax.experimental.pallas import tpu_sc as plsc

N_NODES = 10000
N_EDGES = 320000
D_IN = 128
D_HID = 128
D_OUT = 64

N_TILES = 16
N_PAD = 10112
ROWS_PER_TILE = N_PAD // N_TILES
STEP = 128
TOT_STEPS = 2560
E_PAD = TOT_STEPS * STEP
DEG_SPLIT = 80
ROW_BLK = 2000


def _make_sc_aggregate(width, with_deg, edge_split, s_t, ph, nbuf, sdepth):
  n_ph = s_t // ph
  out_type = [jax.ShapeDtypeStruct((N_PAD, width), jnp.float32),
              jax.ShapeDtypeStruct((N_PAD, width), jnp.float32)]
  scratch = [pltpu.VMEM((ph, STEP), jnp.int32),
             pltpu.VMEM((ph, STEP), jnp.int32),
             pltpu.VMEM((nbuf, STEP, width), jnp.float32),
             pltpu.VMEM_SHARED((N_PAD, width), jnp.float32),
             pltpu.SemaphoreType.DMA,
             pltpu.SemaphoreType.DMA]
  if with_deg:
    out_type += [jax.ShapeDtypeStruct((N_PAD, 16), jnp.float32),
                 jax.ShapeDtypeStruct((N_PAD, 16), jnp.float32)]
    scratch += [pltpu.VMEM((STEP, 16), jnp.float32),
                pltpu.VMEM_SHARED((N_PAD, 16), jnp.float32),
                pltpu.SemaphoreType.DMA]

  mesh = plsc.VectorSubcoreMesh(core_axis_name="c", subcore_axis_name="s")

  def body(*refs):
    if with_deg:
      (ytab, src_hbm, dst_hbm, zf_hbm, zd_hbm, ones_hbm,
       out_a, out_b, deg_a, deg_b,
       src_v, dst_v, rows_v, agg_sh, gsem, ssem, ones_v, deg_sh, dsem) = refs
    else:
      (ytab, src_hbm, dst_hbm, zf_hbm,
       out_a, out_b,
       src_v, dst_v, rows_v, agg_sh, gsem, ssem) = refs
    c = lax.axis_index("c")
    s = lax.axis_index("s")
    tile_rows = pl.ds(s * ROWS_PER_TILE, ROWS_PER_TILE)
    if edge_split:
      tab = ytab
      idx_base = (c * N_TILES + s) * s_t
    else:
      tab = ytab.at[c]
      idx_base = s * s_t

    pltpu.sync_copy(zf_hbm.at[tile_rows], agg_sh.at[tile_rows])
    if with_deg:
      pltpu.sync_copy(zd_hbm.at[tile_rows], deg_sh.at[tile_rows])
      pltpu.sync_copy(ones_hbm, ones_v)
    plsc.subcore_barrier()

    def do_phase(p, carry):
      pltpu.sync_copy(src_hbm.at[pl.ds(idx_base + p * ph, ph)], src_v)
      pltpu.sync_copy(dst_hbm.at[pl.ds(idx_base + p * ph, ph)], dst_v)
      if with_deg and not edge_split:
        cnt = jnp.where(c == 0, p * ph < DEG_SPLIT, p * ph >= DEG_SPLIT)

      for b in range(sdepth):
        pltpu.async_copy(tab.at[src_v.at[b]], rows_v.at[b], gsem)

      def step(j, carry2):
        buf = rows_v.at[j % nbuf]
        pltpu.make_async_copy(tab.at[src_v.at[j]], buf, gsem).wait()
        pltpu.async_copy(buf, agg_sh.at[dst_v.at[j]], ssem, add=True)
        if with_deg:
          if edge_split:
            pltpu.async_copy(ones_v, deg_sh.at[dst_v.at[j]], dsem, add=True)
          else:
            @pl.when(cnt)
            def _():
              pltpu.async_copy(ones_v, deg_sh.at[dst_v.at[j]], dsem, add=True)

        @pl.when(j >= sdepth)
        def _():
          pltpu.make_async_copy(rows_v.at[(j - sdepth) % nbuf],
                                agg_sh.at[dst_v.at[j - sdepth]], ssem).wait()
          if with_deg:
            if edge_split:
              pltpu.make_async_copy(ones_v, deg_sh.at[dst_v.at[j - sdepth]],
                                    dsem).wait()
            else:
              @pl.when(cnt)
              def _():
                pltpu.make_async_copy(ones_v, deg_sh.at[dst_v.at[j - sdepth]],
                                      dsem).wait()

        @pl.when(j + sdepth < ph)
        def _():
          pltpu.async_copy(tab.at[src_v.at[j + sdepth]],
                           rows_v.at[(j + sdepth) % nbuf], gsem)
        return carry2

      lax.fori_loop(0, ph, step, 0)
      for j in range(ph - sdepth, ph):
        pltpu.make_async_copy(rows_v.at[j % nbuf],
                              agg_sh.at[dst_v.at[j]], ssem).wait()
        if with_deg:
          if edge_split:
            pltpu.make_async_copy(ones_v, deg_sh.at[dst_v.at[j]], dsem).wait()
          else:
            @pl.when(cnt)
            def _():
              pltpu.make_async_copy(ones_v, deg_sh.at[dst_v.at[j]],
                                    dsem).wait()
      return carry

    lax.fori_loop(0, n_ph, do_phase, 0)
    plsc.subcore_barrier()

    @pl.when(c == 0)
    def _():
      pltpu.sync_copy(agg_sh.at[tile_rows], out_a.at[tile_rows])
      if with_deg:
        pltpu.sync_copy(deg_sh.at[tile_rows], deg_a.at[tile_rows])

    @pl.when(c == 1)
    def _():
      pltpu.sync_copy(agg_sh.at[tile_rows], out_b.at[tile_rows])
      if with_deg:
        pltpu.sync_copy(deg_sh.at[tile_rows], deg_b.at[tile_rows])

  return pl.kernel(body, out_type=out_type, mesh=mesh, scratch_types=scratch,
                   compiler_params=pltpu.CompilerParams(use_tc_tiling_on_sc=False))


def _tc_layer1(x, wcat):
  def body(x_ref, w_ref, ytab_ref, z_ref):
    y = jnp.dot(x_ref[...], w_ref[...], preferred_element_type=jnp.float32)
    ytab_ref[0] = y[:, :64]
    ytab_ref[1] = y[:, 64:128]
    z_ref[...] = y[:, 128:]

  return pl.pallas_call(
      body,
      grid=(N_NODES // ROW_BLK,),
      in_specs=[pl.BlockSpec((ROW_BLK, D_IN), lambda i: (i, 0)),
                pl.BlockSpec((D_IN, 2 * D_HID), lambda i: (0, 0))],
      out_specs=[pl.BlockSpec((2, ROW_BLK, 64), lambda i: (0, i, 0)),
                 pl.BlockSpec((ROW_BLK, D_HID), lambda i: (i, 0))],
      out_shape=[jax.ShapeDtypeStruct((2, N_NODES, 64), jnp.float32),
                 jax.ShapeDtypeStruct((N_NODES, D_HID), jnp.float32)],
  )(x, wcat)


def _tc_layer2(agg_a, agg_b, deg_a, deg_b, z1, bl1, wcat, bl2):
  def body(a_ref, b_ref, da_ref, db_ref, z1_ref, bl1_ref, w_ref, bl2_ref,
           y_ref, z2_ref):
    deg = da_ref[:, 0:1] + db_ref[:, 0:1]
    inv = 1.0 / jnp.maximum(deg, 1.0)
    mean = jnp.concatenate([a_ref[...], b_ref[...]], axis=1) * inv
    h = jnp.maximum(mean + bl1_ref[...] + z1_ref[...], 0.0)
    y2 = jnp.dot(h, w_ref[...], preferred_element_type=jnp.float32)
    y_ref[...] = y2[:, :D_OUT]
    z2_ref[...] = y2[:, D_OUT:] + bl2_ref[...]

  return pl.pallas_call(
      body,
      grid=(N_NODES // ROW_BLK,),
      in_specs=[pl.BlockSpec((ROW_BLK, 64), lambda i: (i, 0)),
                pl.BlockSpec((ROW_BLK, 64), lambda i: (i, 0)),
                pl.BlockSpec((ROW_BLK, 16), lambda i: (i, 0)),
                pl.BlockSpec((ROW_BLK, 16), lambda i: (i, 0)),
                pl.BlockSpec((ROW_BLK, D_HID), lambda i: (i, 0)),
                pl.BlockSpec((1, D_HID), lambda i: (0, 0)),
                pl.BlockSpec((D_HID, 2 * D_OUT), lambda i: (0, 0)),
                pl.BlockSpec((1, D_OUT), lambda i: (0, 0))],
      out_specs=[pl.BlockSpec((ROW_BLK, D_OUT), lambda i: (i, 0)),
                 pl.BlockSpec((ROW_BLK, D_OUT), lambda i: (i, 0))],
      out_shape=[jax.ShapeDtypeStruct((N_NODES, D_OUT), jnp.float32),
                 jax.ShapeDtypeStruct((N_NODES, D_OUT), jnp.float32)],
  )(agg_a, agg_b, deg_a, deg_b, z1, bl1, wcat, bl2)


def _tc_out(agg_a, agg_b, deg_a, deg_b, z2):
  def body(a_ref, b_ref, da_ref, db_ref, z2_ref, o_ref):
    deg = da_ref[:, 0:1] + db_ref[:, 0:1]
    inv = 1.0 / jnp.maximum(deg, 1.0)
    o = (a_ref[...] + b_ref[...]) * inv + z2_ref[...]
    m = jnp.max(o, axis=1, keepdims=True)
    e = jnp.exp(o - m)
    o_ref[...] = o - m - jnp.log(jnp.sum(e, axis=1, keepdims=True))

  return pl.pallas_call(
      body,
      grid=(N_NODES // ROW_BLK,),
      in_specs=[pl.BlockSpec((ROW_BLK, D_OUT), lambda i: (i, 0)),
                pl.BlockSpec((ROW_BLK, D_OUT), lambda i: (i, 0)),
                pl.BlockSpec((ROW_BLK, 16), lambda i: (i, 0)),
                pl.BlockSpec((ROW_BLK, 16), lambda i: (i, 0)),
                pl.BlockSpec((ROW_BLK, D_OUT), lambda i: (i, 0))],
      out_specs=pl.BlockSpec((ROW_BLK, D_OUT), lambda i: (i, 0)),
      out_shape=jax.ShapeDtypeStruct((N_NODES, D_OUT), jnp.float32),
  )(agg_a, agg_b, deg_a, deg_b, z2)


_sc_agg1 = _make_sc_aggregate(64, with_deg=True, edge_split=False,
                              s_t=160, ph=80, nbuf=6, sdepth=3)
_sc_agg2 = _make_sc_aggregate(64, with_deg=False, edge_split=True,
                              s_t=80, ph=80, nbuf=8, sdepth=4)


def kernel(x, edge_index, Wl1, bl1, Wr1, Wl2, bl2, Wr2):
  src = edge_index[0].astype(jnp.int32)
  dst = edge_index[1].astype(jnp.int32)
  pad = E_PAD - N_EDGES
  pad_ar = jnp.arange(pad, dtype=jnp.int32)
  pad_dst = N_NODES + pad_ar % (N_PAD - N_NODES)
  src2d = jnp.concatenate([src, pad_ar % N_NODES]).reshape(-1, STEP)
  dst2d = jnp.concatenate([dst, pad_dst]).reshape(-1, STEP)

  zf64 = jnp.zeros((N_PAD, 64), jnp.float32)
  zd = jnp.zeros((N_PAD, 16), jnp.float32)
  ones = jnp.ones((STEP, 16), jnp.float32)

  w1cat = jnp.concatenate([Wl1.T, Wr1.T], axis=1)
  w2cat = jnp.concatenate([Wl2.T, Wr2.T], axis=1)

  ytab1, z1 = _tc_layer1(x, w1cat)
  agg1a, agg1b, dega, degb = _sc_agg1(ytab1, src2d, dst2d, zf64, zd, ones)
  ytab2, z2 = _tc_layer2(agg1a, agg1b, dega, degb,
                         z1, bl1.reshape(1, -1), w2cat, bl2.reshape(1, -1))
  agg2a, agg2b = _sc_agg2(ytab2, src2d, dst2d, zf64)
  return _tc_out(agg2a, agg2b, dega, degb, z2)
